# baseline (speedup 1.0000x reference)
import functools

import jax
import jax.numpy as jnp
from jax.experimental import pallas as pl

_B = 16384
_E = 64
_NCODE = 256
_BETA = 0.25
_BLK = 1024


def _dot(a, b):
    return jax.lax.dot_general(
        a, b, (((1,), (0,)), ((), ())), preferred_element_type=jnp.float32)


def _dot_t(a, b):
    return jax.lax.dot_general(
        a, b, (((1,), (1,)), ((), ())), preferred_element_type=jnp.float32)


def _split_f32(c):
    c1 = c.astype(jnp.bfloat16).astype(jnp.float32)
    r = c - c1
    c2 = r.astype(jnp.bfloat16).astype(jnp.float32)
    return c1, c2, r - c2


def _body(x_ref,
          ew0, eb0, ew1, eb1, ew2, eb2, ew3, eb3,
          dw0, db0, dw1, db1, dw2, db2, dw3, db3,
          ca0, cb0_, cc0, ca1, cb1_, cc1, ca2, cb2_, cc2, ca3, cb3_, cc3,
          out_ref, loss_ref, idx_ref):
    i = pl.program_id(0)
    h = x_ref[...]
    h = jnp.maximum(_dot(h, ew0[...]) + eb0[...], 0.0)
    h = jnp.maximum(_dot(h, ew1[...]) + eb1[...], 0.0)
    h = jnp.maximum(_dot(h, ew2[...]) + eb2[...], 0.0)
    z = _dot(h, ew3[...]) + eb3[...]

    res = z
    xq = jnp.zeros_like(z)
    idx_acc = jnp.zeros((_BLK, 4), jnp.int32)
    lane4 = jax.lax.broadcasted_iota(jnp.int32, (_BLK, 4), 1)
    loss_vec = jnp.zeros((8, 128), jnp.float32)
    loss_rows = jax.lax.broadcasted_iota(jnp.int32, (8, 128), 0)

    for l, (r1, r2, r3) in enumerate(((ca0, cb0_, cc0), (ca1, cb1_, cc1),
                                      (ca2, cb2_, cc2), (ca3, cb3_, cc3))):
        c1 = r1[...]
        c2 = r2[...]
        c3 = r3[...]
        c = (c1 + c2) + c3
        d = (jnp.sum(res * res, axis=1, keepdims=True)
             + jnp.sum(c * c, axis=1)[None, :]) - 2.0 * _dot_t(res, c)
        dmin = jnp.min(d, axis=1, keepdims=True)
        code_iota = jax.lax.broadcasted_iota(jnp.int32, d.shape, 1)
        idx = jnp.min(jnp.where(d <= dmin, code_iota, _NCODE),
                      axis=1, keepdims=True)
        onehot = (code_iota == idx).astype(jnp.float32)
        zq = (_dot(onehot, c1) + _dot(onehot, c2)) + _dot(onehot, c3)
        diff = zq - res
        s = jnp.sum(diff * diff)
        loss_vec = loss_vec + jnp.where(loss_rows == l, s, 0.0)
        xq = xq + zq
        res = res - zq
        idx_acc = jnp.where(lane4 == l, idx, idx_acc)

    g = jnp.maximum(_dot(xq, dw0[...]) + db0[...], 0.0)
    g = jnp.maximum(_dot(g, dw1[...]) + db1[...], 0.0)
    g = jnp.maximum(_dot(g, dw2[...]) + db2[...], 0.0)
    out_ref[...] = _dot(g, dw3[...]) + db3[...]

    idx_ref[...] = idx_acc

    @pl.when(i == 0)
    def _init():
        loss_ref[...] = jnp.zeros_like(loss_ref)

    loss_ref[...] += loss_vec


def kernel(x, enc_W0, enc_b0, enc_W1, enc_b1, enc_W2, enc_b2, enc_W3, enc_b3,
           dec_W0, dec_b0, dec_W1, dec_b1, dec_W2, dec_b2, dec_W3, dec_b3,
           codebook0, codebook1, codebook2, codebook3):
    in_dim = x.shape[1]
    grid = (_B // _BLK,)

    def _full(a):
        return pl.BlockSpec(a.shape, lambda i: (0,) * a.ndim)

    biases = [b.reshape(1, -1) for b in
              (enc_b0, enc_b1, enc_b2, enc_b3, dec_b0, dec_b1, dec_b2, dec_b3)]
    ws = (enc_W0, enc_W1, enc_W2, enc_W3, dec_W0, dec_W1, dec_W2, dec_W3)
    cbs = (codebook0, codebook1, codebook2, codebook3)

    in_specs = [pl.BlockSpec((_BLK, in_dim), lambda i: (i, 0))]
    operands = [x]
    for w, b in zip(ws[:4], biases[:4]):
        in_specs += [_full(w), _full(b)]
        operands += [w, b]
    for w, b in zip(ws[4:], biases[4:]):
        in_specs += [_full(w), _full(b)]
        operands += [w, b]
    for cb in cbs:
        for part in _split_f32(cb):
            in_specs.append(_full(part))
            operands.append(part)

    out, loss_mat, idx = pl.pallas_call(
        _body,
        grid=grid,
        in_specs=in_specs,
        out_specs=[
            pl.BlockSpec((_BLK, in_dim), lambda i: (i, 0)),
            pl.BlockSpec((8, 128), lambda i: (0, 0)),
            pl.BlockSpec((_BLK, 4), lambda i: (i, 0)),
        ],
        out_shape=[
            jax.ShapeDtypeStruct((_B, in_dim), jnp.float32),
            jax.ShapeDtypeStruct((8, 128), jnp.float32),
            jax.ShapeDtypeStruct((_B, 4), jnp.int32),
        ],
    )(*operands)

    sums = loss_mat[:4, 0]
    means = sums / (_B * _E)
    rq_loss = jnp.mean(_BETA * means + means)
    return (out, rq_loss, idx)


# speedup vs baseline: 1.0836x; 1.0836x over previous
import functools

import jax
import jax.numpy as jnp
from jax.experimental import pallas as pl

_B = 16384
_E = 64
_NCODE = 256
_BETA = 0.25
_BLK = 2048


def _dot(a, b):
    return jax.lax.dot_general(
        a, b, (((1,), (0,)), ((), ())), preferred_element_type=jnp.float32)


def _dot_t(a, b):
    return jax.lax.dot_general(
        a, b, (((1,), (1,)), ((), ())), preferred_element_type=jnp.float32)


def _split_f32(c):
    c1 = c.astype(jnp.bfloat16).astype(jnp.float32)
    r = c - c1
    c2 = r.astype(jnp.bfloat16).astype(jnp.float32)
    return c1, c2, r - c2


def _body(x_ref,
          ew0, eb0, ew1, eb1, ew2, eb2, ew3, eb3,
          dw0, db0, dw1, db1, dw2, db2, dw3, db3,
          ca0, cb0_, cc0, ca1, cb1_, cc1, ca2, cb2_, cc2, ca3, cb3_, cc3,
          out_ref, loss_ref, idx_ref):
    i = pl.program_id(0)

    consts = []
    for r1, r2, r3 in ((ca0, cb0_, cc0), (ca1, cb1_, cc1),
                       (ca2, cb2_, cc2), (ca3, cb3_, cc3)):
        c1 = r1[...]
        c2 = r2[...]
        c3 = r3[...]
        c = (c1 + c2) + c3
        cn = jnp.sum(c * c, axis=1)[None, :]
        consts.append((c1, c2, c3, c, cn))

    H = _BLK
    lane4 = jax.lax.broadcasted_iota(jnp.int32, (H, 4), 1)

    def chain(xh):
        h = jnp.maximum(_dot(xh, ew0[...]) + eb0[...], 0.0)
        h = jnp.maximum(_dot(h, ew1[...]) + eb1[...], 0.0)
        h = jnp.maximum(_dot(h, ew2[...]) + eb2[...], 0.0)
        res = _dot(h, ew3[...]) + eb3[...]

        xq = jnp.zeros_like(res)
        idx_acc = jnp.zeros((H, 4), jnp.int32)
        sums = []
        for l, (c1, c2, c3, c, cn) in enumerate(consts):
            d = (jnp.sum(res * res, axis=1, keepdims=True) + cn) \
                - 2.0 * _dot_t(res, c)
            dmin = jnp.min(d, axis=1, keepdims=True)
            code_iota = jax.lax.broadcasted_iota(jnp.int32, d.shape, 1)
            idx = jnp.min(jnp.where(d <= dmin, code_iota, _NCODE),
                          axis=1, keepdims=True)
            onehot = (code_iota == idx).astype(jnp.float32)
            zq = (_dot(onehot, c1) + _dot(onehot, c2)) + _dot(onehot, c3)
            diff = zq - res
            sums.append(jnp.sum(diff * diff))
            xq = xq + zq
            res = res - zq
            idx_acc = jnp.where(lane4 == l, idx, idx_acc)

        g = jnp.maximum(_dot(xq, dw0[...]) + db0[...], 0.0)
        g = jnp.maximum(_dot(g, dw1[...]) + db1[...], 0.0)
        g = jnp.maximum(_dot(g, dw2[...]) + db2[...], 0.0)
        return _dot(g, dw3[...]) + db3[...], idx_acc, sums

    out_a, idx_a, sums_a = chain(x_ref[...])
    out_ref[...] = out_a
    idx_ref[...] = idx_a

    loss_rows = jax.lax.broadcasted_iota(jnp.int32, (8, 128), 0)
    loss_vec = jnp.zeros((8, 128), jnp.float32)
    for l in range(4):
        loss_vec = loss_vec + jnp.where(loss_rows == l, sums_a[l], 0.0)

    @pl.when(i == 0)
    def _init():
        loss_ref[...] = jnp.zeros_like(loss_ref)

    loss_ref[...] += loss_vec


def kernel(x, enc_W0, enc_b0, enc_W1, enc_b1, enc_W2, enc_b2, enc_W3, enc_b3,
           dec_W0, dec_b0, dec_W1, dec_b1, dec_W2, dec_b2, dec_W3, dec_b3,
           codebook0, codebook1, codebook2, codebook3):
    in_dim = x.shape[1]
    grid = (_B // _BLK,)

    def _full(a):
        return pl.BlockSpec(a.shape, lambda i: (0,) * a.ndim)

    biases = [b.reshape(1, -1) for b in
              (enc_b0, enc_b1, enc_b2, enc_b3, dec_b0, dec_b1, dec_b2, dec_b3)]
    ws = (enc_W0, enc_W1, enc_W2, enc_W3, dec_W0, dec_W1, dec_W2, dec_W3)
    cbs = (codebook0, codebook1, codebook2, codebook3)

    in_specs = [pl.BlockSpec((_BLK, in_dim), lambda i: (i, 0))]
    operands = [x]
    for w, b in zip(ws[:4], biases[:4]):
        in_specs += [_full(w), _full(b)]
        operands += [w, b]
    for w, b in zip(ws[4:], biases[4:]):
        in_specs += [_full(w), _full(b)]
        operands += [w, b]
    for cb in cbs:
        for part in _split_f32(cb):
            in_specs.append(_full(part))
            operands.append(part)

    out, loss_mat, idx = pl.pallas_call(
        _body,
        grid=grid,
        in_specs=in_specs,
        out_specs=[
            pl.BlockSpec((_BLK, in_dim), lambda i: (i, 0)),
            pl.BlockSpec((8, 128), lambda i: (0, 0)),
            pl.BlockSpec((_BLK, 4), lambda i: (i, 0)),
        ],
        out_shape=[
            jax.ShapeDtypeStruct((_B, in_dim), jnp.float32),
            jax.ShapeDtypeStruct((8, 128), jnp.float32),
            jax.ShapeDtypeStruct((_B, 4), jnp.int32),
        ],
    )(*operands)

    sums = loss_mat[:4, 0]
    means = sums / (_B * _E)
    rq_loss = jnp.mean(_BETA * means + means)
    return (out, rq_loss, idx)


# speedup vs baseline: 1.1818x; 1.0906x over previous
import functools

import jax
import jax.numpy as jnp
from jax.experimental import pallas as pl

_B = 16384
_E = 64
_NCODE = 256
_BETA = 0.25
_BLK = 2048


def _dot(a, b):
    return jax.lax.dot_general(
        a, b, (((1,), (0,)), ((), ())), preferred_element_type=jnp.float32)


def _dot_t(a, b):
    return jax.lax.dot_general(
        a, b, (((1,), (1,)), ((), ())), preferred_element_type=jnp.float32)


def _split_f32(c):
    c1 = c.astype(jnp.bfloat16).astype(jnp.float32)
    r = c - c1
    c2 = r.astype(jnp.bfloat16).astype(jnp.float32)
    return c1, c2, r - c2


def _body(x_ref,
          ew0, eb0, ew1, eb1, ew2, eb2, ew3, eb3,
          dw0, db0, dw1, db1, dw2, db2, dw3, db3,
          ca0, cb0_, cc0, ca1, cb1_, cc1, ca2, cb2_, cc2, ca3, cb3_, cc3,
          out_ref, loss_ref, idx_ref):
    i = pl.program_id(0)

    consts = []
    for r1, r2, r3 in ((ca0, cb0_, cc0), (ca1, cb1_, cc1),
                       (ca2, cb2_, cc2), (ca3, cb3_, cc3)):
        c1 = r1[...]
        c2 = r2[...]
        c3 = r3[...]
        c = (c1 + c2) + c3
        cn = jnp.sum(c * c, axis=1)[None, :]
        consts.append((c1, c2, c3, c, cn))

    H = _BLK
    lane4 = jax.lax.broadcasted_iota(jnp.int32, (H, 4), 1)

    def chain(xh):
        h = jnp.maximum(_dot(xh, ew0[...]) + eb0[...], 0.0)
        h = jnp.maximum(_dot(h, ew1[...]) + eb1[...], 0.0)
        h = jnp.maximum(_dot(h, ew2[...]) + eb2[...], 0.0)
        res = _dot(h, ew3[...]) + eb3[...]

        xq = jnp.zeros_like(res)
        idx_acc = jnp.zeros((H, 4), jnp.int32)
        fiota = jax.lax.broadcasted_iota(
            jnp.int32, (H, _NCODE), 1).astype(jnp.float32)
        sums = []
        for l, (c1, c2, c3, c, cn) in enumerate(consts):
            d = (jnp.sum(res * res, axis=1, keepdims=True) + cn) \
                - 2.0 * _dot_t(res, c)
            dmin = jnp.min(d, axis=1, keepdims=True)
            fidx = jnp.min(jnp.where(d <= dmin, fiota, float(_NCODE)),
                           axis=1, keepdims=True)
            onehot = (fiota == fidx).astype(jnp.float32)
            idx = fidx.astype(jnp.int32)
            zq = (_dot(onehot, c1) + _dot(onehot, c2)) + _dot(onehot, c3)
            diff = zq - res
            sums.append(jnp.sum(diff * diff))
            xq = xq + zq
            res = res - zq
            idx_acc = jnp.where(lane4 == l, idx, idx_acc)

        g = jnp.maximum(_dot(xq, dw0[...]) + db0[...], 0.0)
        g = jnp.maximum(_dot(g, dw1[...]) + db1[...], 0.0)
        g = jnp.maximum(_dot(g, dw2[...]) + db2[...], 0.0)
        return _dot(g, dw3[...]) + db3[...], idx_acc, sums

    out_a, idx_a, sums_a = chain(x_ref[...])
    out_ref[...] = out_a
    idx_ref[...] = idx_a

    loss_rows = jax.lax.broadcasted_iota(jnp.int32, (8, 128), 0)
    loss_vec = jnp.zeros((8, 128), jnp.float32)
    for l in range(4):
        loss_vec = loss_vec + jnp.where(loss_rows == l, sums_a[l], 0.0)

    @pl.when(i == 0)
    def _init():
        loss_ref[...] = jnp.zeros_like(loss_ref)

    loss_ref[...] += loss_vec


def kernel(x, enc_W0, enc_b0, enc_W1, enc_b1, enc_W2, enc_b2, enc_W3, enc_b3,
           dec_W0, dec_b0, dec_W1, dec_b1, dec_W2, dec_b2, dec_W3, dec_b3,
           codebook0, codebook1, codebook2, codebook3):
    in_dim = x.shape[1]
    grid = (_B // _BLK,)

    def _full(a):
        return pl.BlockSpec(a.shape, lambda i: (0,) * a.ndim)

    biases = [b.reshape(1, -1) for b in
              (enc_b0, enc_b1, enc_b2, enc_b3, dec_b0, dec_b1, dec_b2, dec_b3)]
    ws = (enc_W0, enc_W1, enc_W2, enc_W3, dec_W0, dec_W1, dec_W2, dec_W3)
    cbs = (codebook0, codebook1, codebook2, codebook3)

    in_specs = [pl.BlockSpec((_BLK, in_dim), lambda i: (i, 0))]
    operands = [x]
    for w, b in zip(ws[:4], biases[:4]):
        in_specs += [_full(w), _full(b)]
        operands += [w, b]
    for w, b in zip(ws[4:], biases[4:]):
        in_specs += [_full(w), _full(b)]
        operands += [w, b]
    for cb in cbs:
        for part in _split_f32(cb):
            in_specs.append(_full(part))
            operands.append(part)

    out, loss_mat, idx = pl.pallas_call(
        _body,
        grid=grid,
        in_specs=in_specs,
        out_specs=[
            pl.BlockSpec((_BLK, in_dim), lambda i: (i, 0)),
            pl.BlockSpec((8, 128), lambda i: (0, 0)),
            pl.BlockSpec((_BLK, 4), lambda i: (i, 0)),
        ],
        out_shape=[
            jax.ShapeDtypeStruct((_B, in_dim), jnp.float32),
            jax.ShapeDtypeStruct((8, 128), jnp.float32),
            jax.ShapeDtypeStruct((_B, 4), jnp.int32),
        ],
    )(*operands)

    sums = loss_mat[:4, 0]
    means = sums / (_B * _E)
    rq_loss = jnp.mean(_BETA * means + means)
    return (out, rq_loss, idx)


# speedup vs baseline: 1.2517x; 1.0592x over previous
import functools

import jax
import jax.numpy as jnp
from jax.experimental import pallas as pl

_B = 16384
_E = 64
_NCODE = 256
_BETA = 0.25
_BLK = 2048


def _dot(a, b):
    return jax.lax.dot_general(
        a, b, (((1,), (0,)), ((), ())), preferred_element_type=jnp.float32)


def _dot_t(a, b):
    return jax.lax.dot_general(
        a, b, (((1,), (1,)), ((), ())), preferred_element_type=jnp.float32)


def _split_f32(c):
    c1 = c.astype(jnp.bfloat16).astype(jnp.float32)
    r = c - c1
    c2 = r.astype(jnp.bfloat16).astype(jnp.float32)
    return c1, c2, r - c2


def _body(x_ref,
          ew0, eb0, ew1, eb1, ew2, eb2, ew3, eb3,
          dw0, db0, dw1, db1, dw2, db2, dw3, db3,
          ca0, cb0_, cc0, cg0, ca1, cb1_, cc1, cg1,
          ca2, cb2_, cc2, cg2, ca3, cb3_, cc3, cg3,
          tri_ref, icol_ref,
          out_ref, loss_ref, idx_ref):
    i = pl.program_id(0)

    consts = []
    for r1, r2, r3, rg in ((ca0, cb0_, cc0, cg0), (ca1, cb1_, cc1, cg1),
                           (ca2, cb2_, cc2, cg2), (ca3, cb3_, cc3, cg3)):
        c1 = r1[...]
        c2 = r2[...]
        c3 = r3[...]
        c = (c1 + c2) + c3
        cn = jnp.sum(c * c, axis=1)[None, :]
        consts.append((rg, c, cn))
    tri = tri_ref[...]
    icol = icol_ref[...]

    H = _BLK
    lane4 = jax.lax.broadcasted_iota(jnp.int32, (H, 4), 1)

    def chain(xh):
        h = jnp.maximum(_dot(xh, ew0[...]) + eb0[...], 0.0)
        h = jnp.maximum(_dot(h, ew1[...]) + eb1[...], 0.0)
        h = jnp.maximum(_dot(h, ew2[...]) + eb2[...], 0.0)
        res = _dot(h, ew3[...]) + eb3[...]

        xq = jnp.zeros_like(res)
        idx_acc = jnp.zeros((H, 4), jnp.int32)
        sums = []
        for l, (rg, c, cn) in enumerate(consts):
            d = (jnp.sum(res * res, axis=1, keepdims=True) + cn) \
                - 2.0 * _dot_t(res, c)
            dmin = jnp.min(d, axis=1, keepdims=True)
            maskf = (d <= dmin).astype(jnp.float32)
            cnt = _dot(maskf, tri)
            onehot = maskf * jnp.maximum(1.0 - cnt, 0.0)
            idx = _dot(onehot, icol).astype(jnp.int32)
            g = _dot(onehot, rg[...])
            zq = (g[:, 0:_E] + g[:, 128:128 + _E]) + g[:, 256:256 + _E]
            diff = zq - res
            sums.append(jnp.sum(diff * diff))
            xq = xq + zq
            res = res - zq
            idx_acc = jnp.where(lane4 == l, idx, idx_acc)

        g = jnp.maximum(_dot(xq, dw0[...]) + db0[...], 0.0)
        g = jnp.maximum(_dot(g, dw1[...]) + db1[...], 0.0)
        g = jnp.maximum(_dot(g, dw2[...]) + db2[...], 0.0)
        return _dot(g, dw3[...]) + db3[...], idx_acc, sums

    out_a, idx_a, sums_a = chain(x_ref[...])
    out_ref[...] = out_a
    idx_ref[...] = idx_a

    loss_rows = jax.lax.broadcasted_iota(jnp.int32, (8, 128), 0)
    loss_vec = jnp.zeros((8, 128), jnp.float32)
    for l in range(4):
        loss_vec = loss_vec + jnp.where(loss_rows == l, sums_a[l], 0.0)

    @pl.when(i == 0)
    def _init():
        loss_ref[...] = jnp.zeros_like(loss_ref)

    loss_ref[...] += loss_vec


def kernel(x, enc_W0, enc_b0, enc_W1, enc_b1, enc_W2, enc_b2, enc_W3, enc_b3,
           dec_W0, dec_b0, dec_W1, dec_b1, dec_W2, dec_b2, dec_W3, dec_b3,
           codebook0, codebook1, codebook2, codebook3):
    in_dim = x.shape[1]
    grid = (_B // _BLK,)

    def _full(a):
        return pl.BlockSpec(a.shape, lambda i: (0,) * a.ndim)

    biases = [b.reshape(1, -1) for b in
              (enc_b0, enc_b1, enc_b2, enc_b3, dec_b0, dec_b1, dec_b2, dec_b3)]
    ws = (enc_W0, enc_W1, enc_W2, enc_W3, dec_W0, dec_W1, dec_W2, dec_W3)
    cbs = (codebook0, codebook1, codebook2, codebook3)

    in_specs = [pl.BlockSpec((_BLK, in_dim), lambda i: (i, 0))]
    operands = [x]
    for w, b in zip(ws[:4], biases[:4]):
        in_specs += [_full(w), _full(b)]
        operands += [w, b]
    for w, b in zip(ws[4:], biases[4:]):
        in_specs += [_full(w), _full(b)]
        operands += [w, b]
    for cb in cbs:
        parts = _split_f32(cb)
        pad = jnp.zeros((cb.shape[0], 128 - cb.shape[1]), jnp.float32)
        glom = jnp.concatenate(
            [parts[0], pad, parts[1], pad, parts[2], pad], axis=1)
        for part in parts:
            in_specs.append(_full(part))
            operands.append(part)
        in_specs.append(_full(glom))
        operands.append(glom)

    tri = jnp.triu(jnp.ones((256, 256), jnp.float32), k=1)
    icol = jnp.arange(256, dtype=jnp.float32).reshape(256, 1)
    in_specs += [_full(tri), _full(icol)]
    operands += [tri, icol]

    out, loss_mat, idx = pl.pallas_call(
        _body,
        grid=grid,
        in_specs=in_specs,
        out_specs=[
            pl.BlockSpec((_BLK, in_dim), lambda i: (i, 0)),
            pl.BlockSpec((8, 128), lambda i: (0, 0)),
            pl.BlockSpec((_BLK, 4), lambda i: (i, 0)),
        ],
        out_shape=[
            jax.ShapeDtypeStruct((_B, in_dim), jnp.float32),
            jax.ShapeDtypeStruct((8, 128), jnp.float32),
            jax.ShapeDtypeStruct((_B, 4), jnp.int32),
        ],
    )(*operands)

    sums = loss_mat[:4, 0]
    means = sums / (_B * _E)
    rq_loss = jnp.mean(_BETA * means + means)
    return (out, rq_loss, idx)


# speedup vs baseline: 1.3081x; 1.0450x over previous
import functools

import jax
import jax.numpy as jnp
from jax.experimental import pallas as pl

_B = 16384
_E = 64
_NCODE = 256
_BETA = 0.25
_BLK = 2048


def _dot(a, b):
    return jax.lax.dot_general(
        a, b, (((1,), (0,)), ((), ())), preferred_element_type=jnp.float32)


def _dot_t(a, b):
    return jax.lax.dot_general(
        a, b, (((1,), (1,)), ((), ())), preferred_element_type=jnp.float32)


def _split_f32(c):
    c1 = c.astype(jnp.bfloat16).astype(jnp.float32)
    r = c - c1
    c2 = r.astype(jnp.bfloat16).astype(jnp.float32)
    return c1, c2, r - c2


def _body(x_ref,
          ew0, eb0, ew1, eb1, ew2, eb2, ew3, eb3,
          dw0, db0, dw1, db1, dw2, db2, dw3, db3,
          ca0, cb0_, cc0, cg0, ca1, cb1_, cc1, cg1,
          ca2, cb2_, cc2, cg2, ca3, cb3_, cc3, cg3,
          tri_ref, icol_ref,
          out_ref, loss_ref, idx_ref):
    i = pl.program_id(0)

    consts = []
    for r1, r2, r3, rg in ((ca0, cb0_, cc0, cg0), (ca1, cb1_, cc1, cg1),
                           (ca2, cb2_, cc2, cg2), (ca3, cb3_, cc3, cg3)):
        c1 = r1[...]
        c2 = r2[...]
        c3 = r3[...]
        c = (c1 + c2) + c3
        cn = jnp.sum(c * c, axis=1)[None, :]
        consts.append((rg, c, cn))
    tri = tri_ref[...]
    icol = icol_ref[...]

    H = _BLK
    lane4 = jax.lax.broadcasted_iota(jnp.int32, (H, 4), 1)

    def chain(xh):
        h = jnp.maximum(_dot(xh, ew0[...]) + eb0[...], 0.0)
        h = jnp.maximum(_dot(h, ew1[...]) + eb1[...], 0.0)
        h = jnp.maximum(_dot(h, ew2[...]) + eb2[...], 0.0)
        res = _dot(h, ew3[...]) + eb3[...]

        xq = jnp.zeros_like(res)
        idx_acc = jnp.zeros((H, 4), jnp.int32)
        sums = []
        for l, (rg, c, cn) in enumerate(consts):
            d = (jnp.sum(res * res, axis=1, keepdims=True) + cn) \
                - 2.0 * _dot_t(res, c)
            dh = jnp.minimum(d[:, :128], d[:, 128:])
            dmin = jnp.min(dh, axis=1, keepdims=True)
            maskf = (d <= dmin).astype(jnp.float32)
            cnt = _dot(maskf, tri)
            onehot = maskf * jnp.maximum(1.0 - cnt, 0.0)
            g = _dot(onehot, rg[...])
            zq = (g[:, 0:_E] + g[:, 128:128 + _E]) + g[:, 256:256 + _E]
            idx = g[:, 384:385].astype(jnp.int32)
            diff = zq - res
            sums.append(jnp.sum(diff * diff))
            xq = xq + zq
            res = res - zq
            idx_acc = jnp.where(lane4 == l, idx, idx_acc)

        g = jnp.maximum(_dot(xq, dw0[...]) + db0[...], 0.0)
        g = jnp.maximum(_dot(g, dw1[...]) + db1[...], 0.0)
        g = jnp.maximum(_dot(g, dw2[...]) + db2[...], 0.0)
        return _dot(g, dw3[...]) + db3[...], idx_acc, sums

    out_a, idx_a, sums_a = chain(x_ref[...])
    out_ref[...] = out_a
    idx_ref[...] = idx_a

    loss_rows = jax.lax.broadcasted_iota(jnp.int32, (8, 128), 0)
    loss_vec = jnp.zeros((8, 128), jnp.float32)
    for l in range(4):
        loss_vec = loss_vec + jnp.where(loss_rows == l, sums_a[l], 0.0)

    @pl.when(i == 0)
    def _init():
        loss_ref[...] = jnp.zeros_like(loss_ref)

    loss_ref[...] += loss_vec


def kernel(x, enc_W0, enc_b0, enc_W1, enc_b1, enc_W2, enc_b2, enc_W3, enc_b3,
           dec_W0, dec_b0, dec_W1, dec_b1, dec_W2, dec_b2, dec_W3, dec_b3,
           codebook0, codebook1, codebook2, codebook3):
    in_dim = x.shape[1]
    grid = (_B // _BLK,)

    def _full(a):
        return pl.BlockSpec(a.shape, lambda i: (0,) * a.ndim)

    biases = [b.reshape(1, -1) for b in
              (enc_b0, enc_b1, enc_b2, enc_b3, dec_b0, dec_b1, dec_b2, dec_b3)]
    ws = (enc_W0, enc_W1, enc_W2, enc_W3, dec_W0, dec_W1, dec_W2, dec_W3)
    cbs = (codebook0, codebook1, codebook2, codebook3)

    in_specs = [pl.BlockSpec((_BLK, in_dim), lambda i: (i, 0))]
    operands = [x]
    for w, b in zip(ws[:4], biases[:4]):
        in_specs += [_full(w), _full(b)]
        operands += [w, b]
    for w, b in zip(ws[4:], biases[4:]):
        in_specs += [_full(w), _full(b)]
        operands += [w, b]
    icol = jnp.arange(256, dtype=jnp.float32).reshape(256, 1)
    for cb in cbs:
        parts = _split_f32(cb)
        pad = jnp.zeros((cb.shape[0], 128 - cb.shape[1]), jnp.float32)
        glom = jnp.concatenate(
            [parts[0], pad, parts[1], pad, parts[2], pad, icol], axis=1)
        for part in parts:
            in_specs.append(_full(part))
            operands.append(part)
        in_specs.append(_full(glom))
        operands.append(glom)

    tri = jnp.triu(jnp.ones((256, 256), jnp.float32), k=1)
    in_specs += [_full(tri), _full(icol)]
    operands += [tri, icol]

    out, loss_mat, idx = pl.pallas_call(
        _body,
        grid=grid,
        in_specs=in_specs,
        out_specs=[
            pl.BlockSpec((_BLK, in_dim), lambda i: (i, 0)),
            pl.BlockSpec((8, 128), lambda i: (0, 0)),
            pl.BlockSpec((_BLK, 4), lambda i: (i, 0)),
        ],
        out_shape=[
            jax.ShapeDtypeStruct((_B, in_dim), jnp.float32),
            jax.ShapeDtypeStruct((8, 128), jnp.float32),
            jax.ShapeDtypeStruct((_B, 4), jnp.int32),
        ],
    )(*operands)

    sums = loss_mat[:4, 0]
    means = sums / (_B * _E)
    rq_loss = jnp.mean(_BETA * means + means)
    return (out, rq_loss, idx)
